# baseline (speedup 1.0000x reference)
import functools

import jax
import jax.numpy as jnp
from jax import lax
from jax.experimental import pallas as pl
from jax.experimental.pallas import tpu as pltpu, tpu_sc as plsc

B_TOTAL = 16384 * 50
D = 64
NW = 32
B_PER_W = B_TOTAL // NW
CH = 512
N_CHUNKS = B_PER_W // CH
IDX_PER_STREAM = 512
STREAMS_PER_CHUNK = CH // IDX_PER_STREAM

_mesh = plsc.VectorSubcoreMesh(core_axis_name="c", subcore_axis_name="s")


@functools.partial(
    pl.kernel,
    out_type=jax.ShapeDtypeStruct((B_TOTAL, D), jnp.float32),
    mesh=_mesh,
    compiler_params=pltpu.CompilerParams(use_tc_tiling_on_sc=False),
    scratch_types=[
        pltpu.VMEM((B_PER_W,), jnp.int32),
        pltpu.VMEM((CH, D), jnp.float32),
        pltpu.VMEM((D,), jnp.float32),
        pltpu.SemaphoreType.DMA,
    ],
)
def _emb_kernel(x_hbm, w_hbm, b_hbm, out_hbm, idx_v, rows_v, bias_v, sem):
    wid = lax.axis_index("s") * 2 + lax.axis_index("c")
    base = wid * B_PER_W

    pltpu.sync_copy(b_hbm, bias_v)
    pltpu.sync_copy(x_hbm.at[pl.ds(base, B_PER_W)], idx_v)

    bias_regs = [bias_v[pl.ds(16 * j, 16)] for j in range(D // 16)]

    def chunk_body(g, carry):
        copies = []
        for j in range(STREAMS_PER_CHUNK):
            copies.append(
                pltpu.async_copy(
                    w_hbm.at[idx_v.at[pl.ds(g * CH + j * IDX_PER_STREAM,
                                            IDX_PER_STREAM)]],
                    rows_v.at[pl.ds(j * IDX_PER_STREAM, IDX_PER_STREAM)],
                    sem,
                )
            )
        for c in copies:
            c.wait()

        def row_body(r, carry2):
            for j in range(D // 16):
                plsc.addupdate(rows_v.at[r, pl.ds(16 * j, 16)], bias_regs[j])
            return carry2

        lax.fori_loop(0, CH, row_body, 0, unroll=4)

        pltpu.sync_copy(rows_v, out_hbm.at[pl.ds(base + g * CH, CH)])
        return carry

    lax.fori_loop(0, N_CHUNKS, chunk_body, 0)


def kernel(x, weight, bias):
    out = _emb_kernel(x.reshape(-1), weight, bias)
    return out.reshape(x.shape[0], x.shape[1], D)


# speedup vs baseline: 1.0452x; 1.0452x over previous
import functools

import jax
import jax.numpy as jnp
from jax import lax
from jax.experimental import pallas as pl
from jax.experimental.pallas import tpu as pltpu, tpu_sc as plsc

B_TOTAL = 16384 * 50
D = 64
NW = 32
B_PER_W = B_TOTAL // NW
CH = 512
N_CHUNKS = B_PER_W // CH
IDX_PER_STREAM = 512
STREAMS_PER_CHUNK = CH // IDX_PER_STREAM

_mesh = plsc.VectorSubcoreMesh(core_axis_name="c", subcore_axis_name="s")


@functools.partial(
    pl.kernel,
    out_type=jax.ShapeDtypeStruct((B_TOTAL, D), jnp.float32),
    mesh=_mesh,
    compiler_params=pltpu.CompilerParams(use_tc_tiling_on_sc=False),
    scratch_types=[
        pltpu.VMEM((B_PER_W,), jnp.int32),
        pltpu.VMEM((CH, D), jnp.float32),
        pltpu.VMEM((D,), jnp.float32),
        pltpu.SemaphoreType.DMA,
    ],
)
def _emb_kernel(x_hbm, w_hbm, b_hbm, out_hbm, idx_v, rows_v, bias_v, sem):
    wid = lax.axis_index("s") * 2 + lax.axis_index("c")
    base = wid * B_PER_W

    pltpu.sync_copy(b_hbm, bias_v)
    pltpu.sync_copy(x_hbm.at[pl.ds(base, B_PER_W)], idx_v)

    bias_regs = [bias_v[pl.ds(16 * j, 16)] for j in range(D // 16)]

    def chunk_body(g, carry):
        copies = []
        for j in range(STREAMS_PER_CHUNK):
            copies.append(
                pltpu.async_copy(
                    w_hbm.at[idx_v.at[pl.ds(g * CH + j * IDX_PER_STREAM,
                                            IDX_PER_STREAM)]],
                    rows_v.at[pl.ds(j * IDX_PER_STREAM, IDX_PER_STREAM)],
                    sem,
                )
            )
        for c in copies:
            c.wait()

        def row_body(r, carry2):
            for j in range(D // 16):
                plsc.addupdate(rows_v.at[r, pl.ds(16 * j, 16)], bias_regs[j])
            return carry2


        pltpu.sync_copy(rows_v, out_hbm.at[pl.ds(base + g * CH, CH)])
        return carry

    lax.fori_loop(0, N_CHUNKS, chunk_body, 0)


def kernel(x, weight, bias):
    out = _emb_kernel(x.reshape(-1), weight, bias)
    return out.reshape(x.shape[0], x.shape[1], D)


# speedup vs baseline: 1.0647x; 1.0187x over previous
import functools

import jax
import jax.numpy as jnp
from jax import lax
from jax.experimental import pallas as pl
from jax.experimental.pallas import tpu as pltpu, tpu_sc as plsc

B_TOTAL = 16384 * 50
D = 64
NW = 32
B_PER_W = B_TOTAL // NW
CH = 400
N_CHUNKS = B_PER_W // CH
NBUF = 4

_mesh = plsc.VectorSubcoreMesh(core_axis_name="c", subcore_axis_name="s")


@functools.partial(
    pl.kernel,
    out_type=jax.ShapeDtypeStruct((B_TOTAL, D), jnp.float32),
    mesh=_mesh,
    compiler_params=pltpu.CompilerParams(use_tc_tiling_on_sc=False),
    scratch_types=[
        pltpu.VMEM((B_PER_W,), jnp.int32),
        [pltpu.VMEM((CH, D), jnp.float32)] * NBUF,
        pltpu.VMEM((D,), jnp.float32),
        [pltpu.SemaphoreType.DMA] * NBUF,
        [pltpu.SemaphoreType.DMA] * NBUF,
    ],
)
def _emb_kernel(x_hbm, w_hbm, b_hbm, out_hbm, idx_v, rows, bias_v, sem_g,
                sem_s):
    wid = lax.axis_index("s") * 2 + lax.axis_index("c")
    base = wid * B_PER_W

    pltpu.sync_copy(b_hbm, bias_v)
    pltpu.sync_copy(x_hbm.at[pl.ds(base, B_PER_W)], idx_v)

    bias_regs = [bias_v[pl.ds(16 * j, 16)] for j in range(D // 16)]

    def fire_gather(chunk, b):
        pltpu.async_copy(
            w_hbm.at[idx_v.at[pl.ds(chunk * CH, CH)]], rows[b], sem_g[b])

    def fire_store(chunk, b):
        pltpu.async_copy(
            rows[b], out_hbm.at[pl.ds(base + chunk * CH, CH)], sem_s[b])

    def wait_gather(b):
        pltpu.make_async_copy(
            w_hbm.at[pl.ds(0, CH)], rows[b], sem_g[b]).wait()

    def wait_store(b):
        pltpu.make_async_copy(
            rows[b], out_hbm.at[pl.ds(base, CH)], sem_s[b]).wait()

    def bias_add(b):
        def row_body(r, carry):
            for j in range(D // 16):
                plsc.addupdate(rows[b].at[r, pl.ds(16 * j, 16)], bias_regs[j])
            return carry

        lax.fori_loop(0, CH, row_body, 0, unroll=4)

    fire_gather(0, 0)
    fire_gather(1, 1)
    fire_store(2, 2)
    fire_store(3, 3)

    def outer(p, carry):
        for b in range(NBUF):
            g = p * NBUF + b
            b2 = (b + 2) % NBUF
            wait_store(b2)
            fire_gather(jnp.minimum(g + 2, N_CHUNKS - 1), b2)
            wait_gather(b)
            bias_add(b)
            fire_store(g, b)
        return carry

    lax.fori_loop(0, N_CHUNKS // NBUF, outer, 0)

    wait_gather(0)
    wait_gather(1)
    wait_store(2)
    wait_store(3)


def kernel(x, weight, bias):
    out = _emb_kernel(x.reshape(-1), weight, bias)
    return out.reshape(x.shape[0], x.shape[1], D)
